# Initial kernel scaffold; baseline (speedup 1.0000x reference)
import jax
import jax.numpy as jnp
from jax.experimental import pallas as pl


def kernel(x, token_table, pos_table):
    raise NotImplementedError("write your pallas kernel here")


# speedup vs baseline: 1.4240x; 1.4240x over previous
import functools

import jax
import jax.numpy as jnp
from jax import lax
from jax.experimental import pallas as pl
from jax.experimental.pallas import tpu as pltpu
from jax.experimental.pallas import tpu_sc as plsc

VOCAB = 1000000
L = 200
D = 32
B = 4096
N = B * L

NC, NS = 2, 16
NW = NC * NS
ROWS_PER_W = N // NW
CHUNK = 1600
GROUP = 100
NGROUP = CHUNK // GROUP
NCHUNK = ROWS_PER_W // CHUNK
IDX_ROWS = N // GROUP


@functools.partial(
    pl.kernel,
    mesh=plsc.VectorSubcoreMesh(core_axis_name="c", subcore_axis_name="s"),
    out_type=jax.ShapeDtypeStruct((N, D), jnp.float32),
    scratch_types=[
        pltpu.VMEM((NGROUP, GROUP), jnp.int32),
        pltpu.VMEM((CHUNK, D), jnp.float32),
        pltpu.VMEM((L, D), jnp.float32),
        pltpu.SemaphoreType.DMA,
    ],
    compiler_params=pltpu.CompilerParams(use_tc_tiling_on_sc=False),
)
def _emb_kernel(x_hbm, tok_hbm, pos_hbm, out_hbm, idx_v, rows_v, pos_v, sem):
    wid = lax.axis_index("s") * NC + lax.axis_index("c")
    pltpu.sync_copy(pos_hbm, pos_v)

    def chunk_body(c, carry):
        irow0 = wid * (ROWS_PER_W // GROUP) + c * NGROUP
        row0 = wid * ROWS_PER_W + c * CHUNK
        pltpu.sync_copy(x_hbm.at[pl.ds(irow0, NGROUP)], idx_v)
        copies = [
            pltpu.async_copy(
                tok_hbm.at[idx_v.at[g]],
                rows_v.at[pl.ds(g * GROUP, GROUP)],
                sem,
            )
            for g in range(NGROUP)
        ]
        for cp in copies:
            cp.wait()

        def add_body(i, acc):
            for k in range(2):
                pe = pos_v[i, pl.ds(k * 16, 16)]
                po = pos_v[i + GROUP, pl.ds(k * 16, 16)]
                for g in range(NGROUP):
                    plsc.addupdate(
                        rows_v.at[g * GROUP + i, pl.ds(k * 16, 16)],
                        pe if g % 2 == 0 else po,
                    )
            return acc

        lax.fori_loop(0, GROUP, add_body, 0)
        pltpu.sync_copy(rows_v, out_hbm.at[pl.ds(row0, CHUNK)])
        return carry

    lax.fori_loop(0, NCHUNK, chunk_body, 0)


def kernel(x, token_table, pos_table):
    x2d = x.astype(jnp.int32).reshape(IDX_ROWS, GROUP)
    out = _emb_kernel(x2d, token_table, pos_table)
    return out.reshape(B, L, D)


# speedup vs baseline: 1.4744x; 1.0354x over previous
import functools

import jax
import jax.numpy as jnp
from jax import lax
from jax.experimental import pallas as pl
from jax.experimental.pallas import tpu as pltpu
from jax.experimental.pallas import tpu_sc as plsc

VOCAB = 1000000
L = 200
D = 32
B = 4096
N = B * L

NC, NS = 2, 16
NW = NC * NS
ROWS_PER_W = N // NW
CHUNK = 1600
GROUP = 100
NGROUP = CHUNK // GROUP
NCHUNK = ROWS_PER_W // CHUNK
IDX_ROWS = N // GROUP


@functools.partial(
    pl.kernel,
    mesh=plsc.VectorSubcoreMesh(core_axis_name="c", subcore_axis_name="s"),
    out_type=jax.ShapeDtypeStruct((N, D), jnp.float32),
    scratch_types=[
        pltpu.VMEM((2, NGROUP, GROUP), jnp.int32),
        pltpu.VMEM((2, CHUNK, D), jnp.float32),
        pltpu.VMEM((L, D), jnp.float32),
        pltpu.SemaphoreType.DMA,
        pltpu.SemaphoreType.DMA,
        pltpu.SemaphoreType.DMA,
        pltpu.SemaphoreType.DMA,
    ],
    compiler_params=pltpu.CompilerParams(use_tc_tiling_on_sc=False),
)
def _emb_kernel(x_hbm, tok_hbm, pos_hbm, out_hbm, idx_v, rows_v, pos_v,
                gsem0, gsem1, ssem0, ssem1):
    wid = lax.axis_index("s") * NC + lax.axis_index("c")
    gsem = (gsem0, gsem1)
    ssem = (ssem0, ssem1)
    pltpu.sync_copy(pos_hbm, pos_v)

    def fire_chunk(c, buf):
        irow0 = wid * (ROWS_PER_W // GROUP) + c * NGROUP
        pltpu.sync_copy(x_hbm.at[pl.ds(irow0, NGROUP)], idx_v.at[buf])
        return [
            pltpu.async_copy(
                tok_hbm.at[idx_v.at[buf, g]],
                rows_v.at[buf, pl.ds(g * GROUP, GROUP)],
                gsem[buf],
            )
            for g in range(NGROUP)
        ]

    def add_pos(buf):
        def add_body(i, acc):
            for k in range(2):
                pe = pos_v[i, pl.ds(k * 16, 16)]
                po = pos_v[i + GROUP, pl.ds(k * 16, 16)]
                for g in range(NGROUP):
                    plsc.addupdate(
                        rows_v.at[buf, g * GROUP + i, pl.ds(k * 16, 16)],
                        pe if g % 2 == 0 else po,
                    )
            return acc

        lax.fori_loop(0, GROUP, add_body, 0)

    stores = [None, None]
    gathers = fire_chunk(0, 0)
    for c in range(NCHUNK):
        buf = c & 1
        for cp in gathers:
            cp.wait()
        if c + 1 < NCHUNK:
            nbuf = buf ^ 1
            if stores[nbuf] is not None:
                stores[nbuf].wait()
            next_gathers = fire_chunk(c + 1, nbuf)
        add_pos(buf)
        row0 = wid * ROWS_PER_W + c * CHUNK
        stores[buf] = pltpu.async_copy(
            rows_v.at[buf], out_hbm.at[pl.ds(row0, CHUNK)], ssem[buf])
        if c + 1 < NCHUNK:
            gathers = next_gathers
    stores[0].wait()
    stores[1].wait()


def kernel(x, token_table, pos_table):
    x2d = x.astype(jnp.int32).reshape(IDX_ROWS, GROUP)
    out = _emb_kernel(x2d, token_table, pos_table)
    return out.reshape(B, L, D)
